# baseline (speedup 1.0000x reference)
import functools

import jax
import jax.numpy as jnp
from jax import lax
from jax.experimental import pallas as pl
from jax.experimental.pallas import tpu as pltpu
from jax.experimental.pallas import tpu_sc as plsc

_NC = 2
_NS = 16
_NW = _NC * _NS
_LANES = 16


_ROUND = 128


@functools.cache
def _sc_gather(Bn, Dn, rows):
    b_per_w = rows // _NW
    round_ = min(_ROUND, b_per_w)
    n_rounds = b_per_w // round_

    @functools.partial(
        pl.kernel,
        mesh=plsc.VectorSubcoreMesh(core_axis_name="c", subcore_axis_name="s"),
        out_type=jax.ShapeDtypeStruct((rows,), jnp.float32),
        compiler_params=pltpu.CompilerParams(
            use_tc_tiling_on_sc=True, needs_layout_passes=False
        ),
        scratch_types=[
            pltpu.VMEM((b_per_w,), jnp.int32),
            pltpu.VMEM((round_, _LANES), jnp.float32),
            pltpu.VMEM((b_per_w,), jnp.float32),
            pltpu.VMEM((_LANES,), jnp.float32),
            pltpu.VMEM((_LANES,), jnp.int32),
            pltpu.SemaphoreType.DMA,
        ],
    )
    def gather(x_hbm, idx_hbm, scale_hbm, off_hbm, v_hbm, idx_v, bufs, vals, sv, ov, sem):
        wid = lax.axis_index("s") * _NC + lax.axis_index("c")
        base = wid * b_per_w
        pltpu.sync_copy(scale_hbm, sv)
        scale = sv[...]
        pltpu.sync_copy(off_hbm, ov)
        off = ov[...][0]
        pltpu.sync_copy(
            idx_hbm.at[pl.ds(pl.multiple_of(off + base, 8), b_per_w)], idx_v
        )
        lanes_i = lax.iota(jnp.int32, _LANES)
        for rd in range(n_rounds):
            r0 = rd * round_
            handles = []
            for g in range(round_ // _LANES):
                cvec = (idx_v[pl.ds(r0 + g * _LANES, _LANES)] >> 4) << 4
                for j in range(_LANES):
                    i = g * _LANES + j
                    t = r0 + i
                    c0 = pl.multiple_of(cvec[j], 16)
                    row = off + base + t
                    h = pltpu.async_copy(
                        x_hbm.at[row, pl.ds(c0, _LANES)],
                        bufs.at[i],
                        sem,
                    )
                    handles.append(h)
            for h in handles:
                h.wait()
            for g in range(round_ // _LANES):
                sl = pl.ds(r0 + g * _LANES, _LANES)
                lane = idx_v[sl] & (_LANES - 1)
                tidx = lanes_i + g * _LANES
                x_sel = plsc.load_gather(bufs, [tidx, lane])
                vals[sl] = jnp.maximum(x_sel * scale, 0.0)
        pltpu.sync_copy(vals, v_hbm.at[pl.ds(base, b_per_w)])

    return gather


def _merge_body(x_ref, idx_ref, v_ref, o_ref):
    x = x_ref[...]
    idx = idx_ref[0, 0, :]
    v = v_ref[0, 0, :]
    col = lax.broadcasted_iota(jnp.int32, x.shape, 1)
    mask = col == idx[:, None]
    o_ref[...] = jnp.where(mask, v[:, None], jnp.maximum(x, 0.0))


def _merge_body_acc(x_ref, idx_ref, v_ref, y_ref, o_ref):
    del y_ref
    _merge_body(x_ref, idx_ref, v_ref, o_ref)


@functools.cache
def _dense_merge_chunk(Bn, Dn, block_rows, chunk_blocks, blk_off, aliased):
    x_spec = pl.BlockSpec((block_rows, Dn), lambda i: (i + blk_off, 0))
    i_spec = pl.BlockSpec((1, 1, block_rows), lambda i: (i + blk_off, 0, 0))
    v_spec = pl.BlockSpec((1, 1, block_rows), lambda i: (i, 0, 0))
    in_specs = [x_spec, i_spec, v_spec]
    body = _merge_body
    io_aliases = {}
    if aliased:
        in_specs.append(pl.BlockSpec(memory_space=pl.ANY))
        body = _merge_body_acc
        io_aliases = {3: 0}
    return pl.pallas_call(
        body,
        grid=(chunk_blocks,),
        in_specs=in_specs,
        out_specs=pl.BlockSpec((block_rows, Dn), lambda i: (i + blk_off, 0)),
        out_shape=jax.ShapeDtypeStruct((Bn, Dn), jnp.float32),
        input_output_aliases=io_aliases,
    )


_CHUNKS = (2048, 3072, 3072)


def kernel(inputs, coeff, idx):
    Bn, Dn = inputs.shape
    block_rows = 512
    nb = Bn // block_rows
    scale = jnp.full((_LANES,), 1.0 - 2.0 * coeff, dtype=jnp.float32)
    vs = []
    r0 = 0
    for rows in _CHUNKS:
        gather = _sc_gather(Bn, Dn, rows)
        vs.append(gather(inputs, idx, scale, jnp.full((_LANES,), r0, jnp.int32)))
        r0 += rows
    idx3 = idx.reshape(nb, 1, block_rows)
    y = None
    blk_off = 0
    for k, rows in enumerate(_CHUNKS):
        nbc = rows // block_rows
        v3 = vs[k].reshape(nbc, 1, block_rows)
        fn = _dense_merge_chunk(Bn, Dn, block_rows, nbc, blk_off, y is not None)
        args = (inputs, idx3, v3) if y is None else (inputs, idx3, v3, y)
        y = fn(*args)
        blk_off += nbc
    return y


# speedup vs baseline: 1.0305x; 1.0305x over previous
import functools

import jax
import jax.numpy as jnp
from jax import lax
from jax.experimental import pallas as pl
from jax.experimental.pallas import tpu as pltpu
from jax.experimental.pallas import tpu_sc as plsc

_NC = 2
_NS = 16
_NW = _NC * _NS
_LANES = 16


_ROUND = 128


@functools.cache
def _sc_gather(Bn, Dn, rows):
    b_per_w = rows // _NW
    round_ = min(_ROUND, b_per_w)
    n_rounds = b_per_w // round_

    @functools.partial(
        pl.kernel,
        mesh=plsc.VectorSubcoreMesh(core_axis_name="c", subcore_axis_name="s"),
        out_type=jax.ShapeDtypeStruct((rows,), jnp.float32),
        compiler_params=pltpu.CompilerParams(
            use_tc_tiling_on_sc=True, needs_layout_passes=False
        ),
        scratch_types=[
            pltpu.VMEM((b_per_w,), jnp.int32),
            pltpu.VMEM((round_, _LANES), jnp.float32),
            pltpu.VMEM((b_per_w,), jnp.float32),
            pltpu.VMEM((_LANES,), jnp.float32),
            pltpu.VMEM((_LANES,), jnp.int32),
            pltpu.SemaphoreType.DMA,
        ],
    )
    def gather(x_hbm, idx_hbm, scale_hbm, off_hbm, v_hbm, idx_v, bufs, vals, sv, ov, sem):
        wid = lax.axis_index("s") * _NC + lax.axis_index("c")
        base = wid * b_per_w
        pltpu.sync_copy(scale_hbm, sv)
        scale = sv[...]
        pltpu.sync_copy(off_hbm, ov)
        off = ov[...][0]
        pltpu.sync_copy(
            idx_hbm.at[pl.ds(pl.multiple_of(off + base, 8), b_per_w)], idx_v
        )
        lanes_i = lax.iota(jnp.int32, _LANES)
        for rd in range(n_rounds):
            r0 = rd * round_
            handles = []
            for g in range(round_ // _LANES):
                cvec = (idx_v[pl.ds(r0 + g * _LANES, _LANES)] >> 4) << 4
                for j in range(_LANES):
                    i = g * _LANES + j
                    t = r0 + i
                    c0 = pl.multiple_of(cvec[j], 16)
                    row = off + base + t
                    h = pltpu.async_copy(
                        x_hbm.at[row, pl.ds(c0, _LANES)],
                        bufs.at[i],
                        sem,
                    )
                    handles.append(h)
            for h in handles:
                h.wait()
            for g in range(round_ // _LANES):
                sl = pl.ds(r0 + g * _LANES, _LANES)
                lane = idx_v[sl] & (_LANES - 1)
                tidx = lanes_i + g * _LANES
                x_sel = plsc.load_gather(bufs, [tidx, lane])
                vals[sl] = jnp.maximum(x_sel * scale, 0.0)
        pltpu.sync_copy(vals, v_hbm.at[pl.ds(base, b_per_w)])

    return gather


def _merge_body(x_ref, idx_ref, v_ref, o_ref):
    x = x_ref[...]
    idx = idx_ref[0, 0, :]
    v = v_ref[0, 0, :]
    col = lax.broadcasted_iota(jnp.int32, x.shape, 1)
    mask = col == idx[:, None]
    o_ref[...] = jnp.where(mask, v[:, None], jnp.maximum(x, 0.0))


def _merge_body_acc(x_ref, idx_ref, v_ref, y_ref, o_ref):
    del y_ref
    _merge_body(x_ref, idx_ref, v_ref, o_ref)


@functools.cache
def _dense_merge_chunk(Bn, Dn, block_rows, chunk_blocks, blk_off, aliased):
    x_spec = pl.BlockSpec((block_rows, Dn), lambda i: (i + blk_off, 0))
    i_spec = pl.BlockSpec((1, 1, block_rows), lambda i: (i + blk_off, 0, 0))
    v_spec = pl.BlockSpec((1, 1, block_rows), lambda i: (i, 0, 0))
    in_specs = [x_spec, i_spec, v_spec]
    body = _merge_body
    io_aliases = {}
    if aliased:
        in_specs.append(pl.BlockSpec(memory_space=pl.ANY))
        body = _merge_body_acc
        io_aliases = {3: 0}
    return pl.pallas_call(
        body,
        grid=(chunk_blocks,),
        in_specs=in_specs,
        out_specs=pl.BlockSpec((block_rows, Dn), lambda i: (i + blk_off, 0)),
        out_shape=jax.ShapeDtypeStruct((Bn, Dn), jnp.float32),
        input_output_aliases=io_aliases,
    )


_CHUNKS = (4096, 4096)


def kernel(inputs, coeff, idx):
    Bn, Dn = inputs.shape
    block_rows = 512
    nb = Bn // block_rows
    scale = jnp.full((_LANES,), 1.0 - 2.0 * coeff, dtype=jnp.float32)
    vs = []
    r0 = 0
    for rows in _CHUNKS:
        gather = _sc_gather(Bn, Dn, rows)
        vs.append(gather(inputs, idx, scale, jnp.full((_LANES,), r0, jnp.int32)))
        r0 += rows
    idx3 = idx.reshape(nb, 1, block_rows)
    y = None
    blk_off = 0
    for k, rows in enumerate(_CHUNKS):
        nbc = rows // block_rows
        v3 = vs[k].reshape(nbc, 1, block_rows)
        fn = _dense_merge_chunk(Bn, Dn, block_rows, nbc, blk_off, y is not None)
        args = (inputs, idx3, v3) if y is None else (inputs, idx3, v3, y)
        y = fn(*args)
        blk_off += nbc
    return y
